# baseline (speedup 1.0000x reference)
import jax
import jax.numpy as jnp
import numpy as np
from jax.experimental import pallas as pl
from jax.experimental.pallas import tpu as pltpu

_IN_CH = 96
_OUT_CH = 192
_HO = 112
_WO = 112
_G = 8

_COEF_TABLE = np.array([
    [0, 0, 0, 0],
    [0, 0, 0, 1],
    [0, 1, 0, -1],
    [0, 1, 0, 0],
    [0, 0, 1, -1],
    [0, 0, 1, 0],
    [0, 1, 1, -2],
    [0, 1, 1, -1],
    [1, -1, -1, 1],
    [1, -1, -1, 2],
    [1, 0, -1, 0],
    [1, 0, -1, 1],
    [1, -1, 0, 0],
    [1, -1, 0, 1],
    [1, 0, 0, -1],
    [1, 0, 0, 0],
], dtype=np.float32)


def _weff_kernel(w_ref, coef_ref, out_ref):
    w = w_ref[...].reshape(_OUT_CH * 4, 16)
    w = jax.nn.softmax(w, axis=-1)
    out_ref[...] = jnp.dot(w, coef_ref[...],
                           preferred_element_type=jnp.float32)


def _fused_kernel(ch_ref, ry_ref, rx_ref, weff_ref, s_ref, x_ref, out_ref):
    o = pl.program_id(1)

    def tap(k):
        kk = o * (8 * _G) + k
        c_k = ch_ref[kk]
        ry = ry_ref[kk]
        rx = rx_ref[kk]
        t = x_ref[0, c_k, ry]
        return jnp.dot(t, s_ref[rx],
                       preferred_element_type=jnp.float32)

    for c in range(4 * _G):
        a = tap(2 * c)
        b = tap(2 * c + 1)
        row = o * (4 * _G) + c
        out_ref[0, c] = (weff_ref[row, 0]
                         + weff_ref[row, 1] * a
                         + weff_ref[row, 2] * b
                         + weff_ref[row, 3] * (a * b))


@jax.jit
def kernel(x, weights, selection):
    n, c, h, w = x.shape

    ch = ((selection >> 16) & 0xFFFF).astype(jnp.int32).reshape(-1)
    ry = ((selection >> 8) & 0xFF).astype(jnp.int32).reshape(-1)
    rx = (selection & 0xFF).astype(jnp.int32).reshape(-1)

    weff = pl.pallas_call(
        _weff_kernel,
        out_shape=jax.ShapeDtypeStruct((4 * _OUT_CH, 4), jnp.float32),
    )(weights, jnp.asarray(_COEF_TABLE))

    ssel = np.zeros((3, w, _WO), dtype=np.float32)
    for d in range(3):
        for j in range(_WO):
            q = 2 * j + d - 1
            if 0 <= q < w:
                ssel[d, q, j] = 1.0
    ssel = ssel.astype(np.float32)

    xr = jnp.pad(x, ((0, 0), (0, 0), (1, 1), (0, 0)))
    xs = jnp.stack([xr[:, :, d:d + 2 * _HO - 1:2, :] for d in range(3)],
                   axis=2).astype(jnp.bfloat16)

    grid_spec = pltpu.PrefetchScalarGridSpec(
        num_scalar_prefetch=4,
        grid=(n, _OUT_CH // _G),
        in_specs=[
            pl.BlockSpec((3, w, _WO), lambda n_, o, *_: (0, 0, 0)),
            pl.BlockSpec((1, c, 3, _HO, w),
                         lambda n_, o, *_: (n_, 0, 0, 0, 0)),
        ],
        out_specs=pl.BlockSpec((1, 4 * _G, _HO, _WO),
                               lambda n_, o, *_: (n_, o, 0, 0)),
    )
    out = pl.pallas_call(
        _fused_kernel,
        grid_spec=grid_spec,
        out_shape=jax.ShapeDtypeStruct((n, 4 * _OUT_CH, _HO, _WO),
                                       jnp.float32),
        compiler_params=pltpu.CompilerParams(
            dimension_semantics=("arbitrary", "arbitrary"),
        ),
    )(ch, ry, rx, weff, jnp.asarray(ssel, dtype=jnp.bfloat16), xs)
    return out
